# baseline (speedup 1.0000x reference)
import jax
import jax.numpy as jnp
import numpy as np
from jax.experimental import pallas as pl

jax.config.update("jax_enable_x64", True)

K = 8192
C = 32
NTOK = 8 * 1024
T = 256
NB = NTOK // T
_I0 = np.int32(0)


def _row_sumsq(x):
    sq = x * x
    acc = ((sq[:, 0:8] + sq[:, 8:16]) + sq[:, 16:24]) + sq[:, 24:32]
    t1 = acc[:, 0:4] + acc[:, 4:8]
    t2 = t1[:, 0:2] + t1[:, 2:4]
    return t2[:, 0:1] + t2[:, 1:2]


def _col_sumsq(xt):
    sq = xt * xt
    acc = ((sq[0:8, :] + sq[8:16, :]) + sq[16:24, :]) + sq[24:32, :]
    t1 = acc[0:4, :] + acc[4:8, :]
    t2 = t1[0:2, :] + t1[2:4, :]
    return t2[0:1, :] + t2[1:2, :]


def _rvq_body(z_ref, cb0_ref, cb1_ref, cb2_ref,
              cbt0_ref, cbt1_ref, cbt2_ref,
              zq_ref, idx0_ref, idx1_ref, idx2_ref, ssq_ref):
    zb = z_ref[...]
    cb_refs = (cb0_ref, cb1_ref, cb2_ref)
    cbt_refs = (cbt0_ref, cbt1_ref, cbt2_ref)
    idx_refs = (idx0_ref, idx1_ref, idx2_ref)
    zq = jnp.zeros_like(zb)
    ssqs = []
    for s in range(3):
        cb = cb_refs[s][...]
        cbsq = _col_sumsq(cbt_refs[s][...])
        r = zb - zq
        rsq = _row_sumsq(r)
        prod = jax.lax.dot_general(
            r, cb, (((1,), (1,)), ((), ())),
            preferred_element_type=jnp.float32)
        dist = (rsq + cbsq) - 2.0 * prod
        idx = jax.lax.argmin(dist, 1, jnp.int32)
        oh = (jax.lax.broadcasted_iota(jnp.int32, (T, K), 1)
              == idx[:, None]).astype(jnp.float32)
        q = jax.lax.dot_general(
            oh, cb, (((1,), (0,)), ((), ())),
            precision=jax.lax.Precision.HIGHEST,
            preferred_element_type=jnp.float32)
        q = r + (q - r)
        zq = zq + q
        r_next = zb - zq
        ssqs.append(jnp.sum(r_next * r_next))
        idx_refs[s][0, 0, :] = idx
    zq_ref[...] = zq
    lane = jax.lax.broadcasted_iota(jnp.int32, (128,), 0)
    vec = jnp.where(lane == 0, ssqs[0],
                    jnp.where(lane == 1, ssqs[1],
                              jnp.where(lane == 2, ssqs[2], 0.0)))
    ssq_ref[0, 0, :] = vec


_rvq = pl.pallas_call(
    _rvq_body,
    grid=(NB,),
    in_specs=[
        pl.BlockSpec((T, C), lambda i: (i, _I0)),
        pl.BlockSpec((K, C), lambda i: (_I0, _I0)),
        pl.BlockSpec((K, C), lambda i: (_I0, _I0)),
        pl.BlockSpec((K, C), lambda i: (_I0, _I0)),
        pl.BlockSpec((C, K), lambda i: (_I0, _I0)),
        pl.BlockSpec((C, K), lambda i: (_I0, _I0)),
        pl.BlockSpec((C, K), lambda i: (_I0, _I0)),
    ],
    out_specs=[
        pl.BlockSpec((T, C), lambda i: (i, _I0)),
        pl.BlockSpec((1, 1, T), lambda i: (i, _I0, _I0)),
        pl.BlockSpec((1, 1, T), lambda i: (i, _I0, _I0)),
        pl.BlockSpec((1, 1, T), lambda i: (i, _I0, _I0)),
        pl.BlockSpec((1, 1, 128), lambda i: (i, _I0, _I0)),
    ],
    out_shape=[
        jax.ShapeDtypeStruct((NTOK, C), jnp.float32),
        jax.ShapeDtypeStruct((NB, 1, T), jnp.int32),
        jax.ShapeDtypeStruct((NB, 1, T), jnp.int32),
        jax.ShapeDtypeStruct((NB, 1, T), jnp.int32),
        jax.ShapeDtypeStruct((NB, 1, 128), jnp.float32),
    ],
)


def kernel(z, cb0, cb1, cb2):
    zf = z.reshape(NTOK, C)
    zq, i0, i1, i2, ssq = _rvq(zf, cb0, cb1, cb2,
                               cb0.T, cb1.T, cb2.T)
    z_q = zq.reshape(z.shape)
    i0 = i0.reshape(NTOK).astype(jnp.int64)
    i1 = i1.reshape(NTOK).astype(jnp.int64)
    i2 = i2.reshape(NTOK).astype(jnp.int64)
    index_sum = (i0 + i1 * K + i2 * (K * K)).reshape(z.shape[:-1])
    n = z.size
    s1 = jnp.sum(ssq[:, 0, 0])
    s2 = jnp.sum(ssq[:, 0, 1])
    s3 = jnp.sum(ssq[:, 0, 2])
    loss = ((2.0 * (s1 + s2 + s3) / n) / 3.0 + s3 / n).astype(jnp.float32)
    return (z_q, index_sum, loss)


# speedup vs baseline: 3.9619x; 3.9619x over previous
import jax
import jax.numpy as jnp
import numpy as np
from jax.experimental import pallas as pl

jax.config.update("jax_enable_x64", True)

K = 8192
C = 32
NTOK = 8 * 1024
T = 256
NB = NTOK // T
_I0 = np.int32(0)


def _row_sumsq(x):
    sq = x * x
    acc = ((sq[:, 0:8] + sq[:, 8:16]) + sq[:, 16:24]) + sq[:, 24:32]
    t1 = acc[:, 0:4] + acc[:, 4:8]
    t2 = t1[:, 0:2] + t1[:, 2:4]
    return t2[:, 0:1] + t2[:, 1:2]


def _col_sumsq(xt):
    sq = xt * xt
    acc = ((sq[0:8, :] + sq[8:16, :]) + sq[16:24, :]) + sq[24:32, :]
    t1 = acc[0:4, :] + acc[4:8, :]
    t2 = t1[0:2, :] + t1[2:4, :]
    return t2[0:1, :] + t2[1:2, :]


def _rvq_body(z_ref, cbd0_ref, cbd1_ref, cbd2_ref,
              cbt0_ref, cbt1_ref, cbt2_ref,
              cbm0_ref, cbm1_ref, cbm2_ref,
              zq_ref, idx0_ref, idx1_ref, idx2_ref, ssq_ref):
    zb = z_ref[...]
    cbd_refs = (cbd0_ref, cbd1_ref, cbd2_ref)
    cbt_refs = (cbt0_ref, cbt1_ref, cbt2_ref)
    cbm_refs = (cbm0_ref, cbm1_ref, cbm2_ref)
    idx_refs = (idx0_ref, idx1_ref, idx2_ref)
    zq = jnp.zeros_like(zb)
    ssqs = []
    for s in range(3):
        cbsq = _col_sumsq(cbt_refs[s][...])
        r = zb - zq
        rsq = _row_sumsq(r)
        prod2 = jax.lax.dot_general(
            r, cbd_refs[s][...], (((1,), (1,)), ((), ())),
            preferred_element_type=jnp.float32)
        dist = (rsq + cbsq) - prod2
        idx = jax.lax.argmin(dist, 1, jnp.int32)
        oh = (jax.lax.broadcasted_iota(jnp.int32, (T, K), 1)
              == idx[:, None]).astype(jnp.bfloat16)
        qp = jax.lax.dot_general(
            oh, cbm_refs[s][...], (((1,), (0,)), ((), ())),
            preferred_element_type=jnp.float32)
        q = (qp[:, 0:C] + qp[:, C:2 * C]) + qp[:, 2 * C:3 * C]
        q = r + (q - r)
        zq = zq + q
        r_next = zb - zq
        ssqs.append(jnp.sum(r_next * r_next))
        idx_refs[s][0, 0, :] = idx
    zq_ref[...] = zq
    lane = jax.lax.broadcasted_iota(jnp.int32, (128,), 0)
    vec = jnp.where(lane == 0, ssqs[0],
                    jnp.where(lane == 1, ssqs[1],
                              jnp.where(lane == 2, ssqs[2], 0.0)))
    ssq_ref[0, 0, :] = vec


_rvq = pl.pallas_call(
    _rvq_body,
    grid=(NB,),
    in_specs=[
        pl.BlockSpec((T, C), lambda i: (i, _I0)),
        pl.BlockSpec((K, C), lambda i: (_I0, _I0)),
        pl.BlockSpec((K, C), lambda i: (_I0, _I0)),
        pl.BlockSpec((K, C), lambda i: (_I0, _I0)),
        pl.BlockSpec((C, K), lambda i: (_I0, _I0)),
        pl.BlockSpec((C, K), lambda i: (_I0, _I0)),
        pl.BlockSpec((C, K), lambda i: (_I0, _I0)),
        pl.BlockSpec((K, 3 * C), lambda i: (_I0, _I0)),
        pl.BlockSpec((K, 3 * C), lambda i: (_I0, _I0)),
        pl.BlockSpec((K, 3 * C), lambda i: (_I0, _I0)),
    ],
    out_specs=[
        pl.BlockSpec((T, C), lambda i: (i, _I0)),
        pl.BlockSpec((1, 1, T), lambda i: (i, _I0, _I0)),
        pl.BlockSpec((1, 1, T), lambda i: (i, _I0, _I0)),
        pl.BlockSpec((1, 1, T), lambda i: (i, _I0, _I0)),
        pl.BlockSpec((1, 1, 128), lambda i: (i, _I0, _I0)),
    ],
    out_shape=[
        jax.ShapeDtypeStruct((NTOK, C), jnp.float32),
        jax.ShapeDtypeStruct((NB, 1, T), jnp.int32),
        jax.ShapeDtypeStruct((NB, 1, T), jnp.int32),
        jax.ShapeDtypeStruct((NB, 1, T), jnp.int32),
        jax.ShapeDtypeStruct((NB, 1, 128), jnp.float32),
    ],
)


def _bf16_split3(cb):
    hi = jax.lax.reduce_precision(cb, 8, 7)
    rem = cb - hi
    mid = jax.lax.reduce_precision(rem, 8, 7)
    lo = rem - mid
    return jnp.concatenate([hi.astype(jnp.bfloat16),
                            mid.astype(jnp.bfloat16),
                            lo.astype(jnp.bfloat16)], axis=1)


def kernel(z, cb0, cb1, cb2):
    zf = z.reshape(NTOK, C)
    zq, i0, i1, i2, ssq = _rvq(
        zf, 2.0 * cb0, 2.0 * cb1, 2.0 * cb2,
        cb0.T, cb1.T, cb2.T,
        _bf16_split3(cb0), _bf16_split3(cb1), _bf16_split3(cb2))
    z_q = zq.reshape(z.shape)
    i0 = i0.reshape(NTOK).astype(jnp.int64)
    i1 = i1.reshape(NTOK).astype(jnp.int64)
    i2 = i2.reshape(NTOK).astype(jnp.int64)
    index_sum = (i0 + i1 * K + i2 * (K * K)).reshape(z.shape[:-1])
    n = z.size
    s1 = jnp.sum(ssq[:, 0, 0])
    s2 = jnp.sum(ssq[:, 0, 1])
    s3 = jnp.sum(ssq[:, 0, 2])
    loss = ((2.0 * (s1 + s2 + s3) / n) / 3.0 + s3 / n).astype(jnp.float32)
    return (z_q, index_sum, loss)
